# baseline (speedup 1.0000x reference)
import jax
import jax.numpy as jnp
from jax import lax
from jax.experimental import pallas as pl
from jax.experimental.pallas import tpu as pltpu
from jax.experimental.pallas import tpu_sc as plsc

N = 10000
E = 320000
H = 128
C_OUT = 2
N_PAD = 10240
E_PAD = 327680
EROWS = E_PAD // 128
NC, NS = 2, 16
ROWS_PER_TILE = N_PAD // NS
R2 = 320
R6 = 400


def _sc_mesh():
    return plsc.VectorSubcoreMesh(
        core_axis_name="c", subcore_axis_name="s",
        num_cores=NC, num_subcores=NS)


RPT = EROWS // NC // NS


def _deg_body(dst2d, ones_in, zeros_in, degp, dstall, ones, y, sem):
    c = lax.axis_index("c")
    s = lax.axis_index("s")
    wid = c * NS + s
    pltpu.sync_copy(dst2d.at[pl.ds(wid * RPT, RPT)], dstall)
    pltpu.sync_copy(zeros_in, ones)
    base_r = s * ROWS_PER_TILE
    for k in range(ROWS_PER_TILE // 128):
        pltpu.sync_copy(ones, y.at[pl.ds(base_r + k * 128, 128)])
    pltpu.sync_copy(ones_in, ones)
    plsc.subcore_barrier()

    def step(j, carry):
        pltpu.sync_copy(ones, y.at[dstall.at[j]], add=True)
        return carry

    lax.fori_loop(0, RPT, step, 0)
    plsc.subcore_barrier()
    for k in range(ROWS_PER_TILE // 128):
        r = base_r + k * 128
        pltpu.sync_copy(y.at[pl.ds(r, 128)], degp.at[c, pl.ds(r, 128)])


_deg_call = pl.kernel(
    _deg_body,
    out_type=jax.ShapeDtypeStruct((NC, N_PAD, H), jnp.float32),
    mesh=_sc_mesh(),
    scratch_types=[
        pltpu.VMEM((RPT, 128), jnp.int32),
        pltpu.VMEM((128, H), jnp.float32),
        pltpu.VMEM_SHARED((N_PAD, H), jnp.float32),
        pltpu.SemaphoreType.DMA,
    ],
)


NBUF = 2


def _seg_body(g, eb, zeros_in, sout, ibuf, rows, y, gsem, ssem):
    c = lax.axis_index("c")
    s = lax.axis_index("s")
    wid = c * NS + s

    pltpu.sync_copy(zeros_in, rows.at[0])
    base_r = s * ROWS_PER_TILE
    zcp = []
    for k in range(ROWS_PER_TILE // 128):
        zcp.append(pltpu.async_copy(
            rows.at[0], y.at[pl.ds(base_r + k * 128, 128)], ssem.at[0]))
    for cp in zcp:
        cp.wait()
    plsc.subcore_barrier()

    nbody = RPT // 8

    def mblock(m, carry):
        pltpu.sync_copy(eb.at[wid * nbody + m], ibuf)
        gcp = [None] * 8
        scp = [None] * 8
        gcp[0] = pltpu.async_copy(g.at[ibuf.at[0]], rows.at[0], gsem.at[0])
        for j in range(8):
            b = j % NBUF
            gcp[j].wait()
            scp[j] = pltpu.async_copy(
                rows.at[b], y.at[ibuf.at[8 + j]], ssem.at[b], add=True)
            if j < 7:
                bn = (j + 1) % NBUF
                if j - (NBUF - 1) >= 0:
                    scp[j - (NBUF - 1)].wait()
                gcp[j + 1] = pltpu.async_copy(
                    g.at[ibuf.at[j + 1]], rows.at[bn], gsem.at[bn])
        for j in range(8 - NBUF, 8):
            scp[j].wait()
        return carry

    lax.fori_loop(0, nbody, mblock, 0)
    plsc.subcore_barrier()
    ocp = []
    for k in range(ROWS_PER_TILE // 128):
        r = base_r + k * 128
        ocp.append(pltpu.async_copy(
            y.at[pl.ds(r, 128)], sout.at[c, pl.ds(r, 128)],
            gsem.at[k % NBUF]))
    for cp in ocp:
        cp.wait()


_seg_call = pl.kernel(
    _seg_body,
    out_type=jax.ShapeDtypeStruct((NC, N_PAD, H), jnp.float32),
    mesh=_sc_mesh(),
    scratch_types=[
        pltpu.VMEM((16, 128), jnp.int32),
        pltpu.VMEM((NBUF, 128, H), jnp.float32),
        pltpu.VMEM_SHARED((N_PAD, H), jnp.float32),
        pltpu.SemaphoreType.DMA((NBUF,)),
        pltpu.SemaphoreType.DMA((NBUF,)),
    ],
)


def _hmlp_body(x_ref, w1_ref, b1_ref, w2_ref, b2_ref, h_ref):
    x = x_ref[...]
    h1 = jnp.maximum(
        jnp.dot(x, w1_ref[...], preferred_element_type=jnp.float32)
        + b1_ref[...], 0.0)
    h_ref[...] = jnp.maximum(
        jnp.dot(h1, w2_ref[...], preferred_element_type=jnp.float32)
        + b2_ref[...], 0.0)


def _hmlp_call(x_pad, w1, b1r, w2, b2r):
    nb = N_PAD // R2
    return pl.pallas_call(
        _hmlp_body,
        grid=(nb,),
        in_specs=[
            pl.BlockSpec((R2, H), lambda i: (i, 0)),
            pl.BlockSpec((H, H), lambda i: (0, 0)),
            pl.BlockSpec((1, H), lambda i: (0, 0)),
            pl.BlockSpec((H, H), lambda i: (0, 0)),
            pl.BlockSpec((1, H), lambda i: (0, 0)),
        ],
        out_specs=pl.BlockSpec((R2, H), lambda i: (i, 0)),
        out_shape=jax.ShapeDtypeStruct((N_PAD, H), jnp.float32),
    )(x_pad, w1, b1r, w2, b2r)


def _scale_body(degp_ref, h_ref, g_ref, dinv_ref):
    i = pl.program_id(0)
    deg = (degp_ref[0, :, 0:1].astype(jnp.float32)
           + degp_ref[1, :, 0:1].astype(jnp.float32))
    dinv = lax.rsqrt(jnp.maximum(deg, 1.0))
    rows = i * R2 + lax.broadcasted_iota(jnp.int32, (R2, 1), 0)
    g = jnp.where(rows < N, h_ref[...] * dinv, 0.0)
    g_ref[...] = g
    dinv_ref[...] = dinv


def _scale_call(degp, h):
    nb = N_PAD // R2
    return pl.pallas_call(
        _scale_body,
        grid=(nb,),
        in_specs=[
            pl.BlockSpec((NC, R2, H), lambda i: (0, i, 0)),
            pl.BlockSpec((R2, H), lambda i: (i, 0)),
        ],
        out_specs=[
            pl.BlockSpec((R2, H), lambda i: (i, 0)),
            pl.BlockSpec((R2, 1), lambda i: (i, 0)),
        ],
        out_shape=[
            jax.ShapeDtypeStruct((N_PAD, H), jnp.float32),
            jax.ShapeDtypeStruct((N_PAD, 1), jnp.float32),
        ],
    )(degp, h)


def _elem_body(h_ref, dinv_ref, s1_ref, f1_ref, g2_ref):
    i = pl.program_id(0)
    h = h_ref[...]
    dinv = dinv_ref[...]
    s1 = s1_ref[0] + s1_ref[1]
    f1 = h - dinv * s1
    rows = i * R2 + lax.broadcasted_iota(jnp.int32, (R2, 1), 0)
    g2 = jnp.where(rows < N, f1 * dinv, 0.0)
    f1_ref[...] = f1
    g2_ref[...] = g2


def _elem_call(h, dinv, s1p):
    nb = N_PAD // R2
    return pl.pallas_call(
        _elem_body,
        grid=(nb,),
        in_specs=[
            pl.BlockSpec((R2, H), lambda i: (i, 0)),
            pl.BlockSpec((R2, 1), lambda i: (i, 0)),
            pl.BlockSpec((NC, R2, H), lambda i: (0, i, 0)),
        ],
        out_specs=[
            pl.BlockSpec((R2, H), lambda i: (i, 0)),
            pl.BlockSpec((R2, H), lambda i: (i, 0)),
        ],
        out_shape=[
            jax.ShapeDtypeStruct((N_PAD, H), jnp.float32),
            jax.ShapeDtypeStruct((N_PAD, H), jnp.float32),
        ],
    )(h, dinv, s1p)


def _final_body(h_ref, f1_ref, dinv_ref, s2_ref, w3_ref, b3_ref, w4_ref,
                b4_ref, o_ref):
    h = h_ref[...]
    f1 = f1_ref[...]
    dinv = dinv_ref[...]
    s2 = s2_ref[0] + s2_ref[1]
    f2 = f1 - dinv * s2
    t0 = 3.0 * h - 3.0 * f1 + 0.75 * f2
    t1 = 3.0 * f1 - 1.5 * f2
    t2 = 0.75 * f2
    hf = jnp.concatenate([t0, t1, t2], axis=1)
    z = jnp.maximum(
        jnp.dot(hf, w3_ref[...], preferred_element_type=jnp.float32)
        + b3_ref[...], 0.0)
    o_ref[...] = (
        jnp.dot(z, w4_ref[...], preferred_element_type=jnp.float32)
        + b4_ref[...])


def _final_call(h, f1, dinv, s2p, w3, b3r, w4, b4r):
    nb = N // R6
    return pl.pallas_call(
        _final_body,
        grid=(nb,),
        in_specs=[
            pl.BlockSpec((R6, H), lambda i: (i, 0)),
            pl.BlockSpec((R6, H), lambda i: (i, 0)),
            pl.BlockSpec((R6, 1), lambda i: (i, 0)),
            pl.BlockSpec((NC, R6, H), lambda i: (0, i, 0)),
            pl.BlockSpec((3 * H, H), lambda i: (0, 0)),
            pl.BlockSpec((1, H), lambda i: (0, 0)),
            pl.BlockSpec((H, C_OUT), lambda i: (0, 0)),
            pl.BlockSpec((1, C_OUT), lambda i: (0, 0)),
        ],
        out_specs=pl.BlockSpec((R6, C_OUT), lambda i: (i, 0)),
        out_shape=jax.ShapeDtypeStruct((N, C_OUT), jnp.float32),
    )(h, f1, dinv, s2p, w3, b3r, w4, b4r)


def kernel(in_feat, edge_index, w1, b1, w2, b2, w3, b3, w4, b4):
    x_pad = jnp.pad(in_feat, ((0, N_PAD - N), (0, 0)))
    nw = NC * NS
    padw = (E_PAD - E) // nw
    pad = jnp.broadcast_to(
        jnp.arange(N, N + padw, dtype=jnp.int32), (nw, padw))
    src_w = jnp.concatenate([edge_index[0].reshape(nw, E // nw), pad], 1)
    dst_w = jnp.concatenate([edge_index[1].reshape(nw, E // nw), pad], 1)
    dst2d = dst_w.reshape(EROWS, 128)
    src4 = src_w.reshape(nw, RPT // 8, 8, 128)
    dst4 = dst_w.reshape(nw, RPT // 8, 8, 128)
    eb = jnp.concatenate([src4, dst4], axis=2).reshape(
        nw * (RPT // 8), 16, 128)
    b1r = b1.reshape(1, H)
    b2r = b2.reshape(1, H)
    b3r = b3.reshape(1, H)
    b4r = b4.reshape(1, C_OUT)

    ones_in = jnp.ones((128, H), dtype=jnp.float32)
    zeros_in = jnp.zeros((128, H), dtype=jnp.float32)
    degp = _deg_call(dst2d, ones_in, zeros_in)
    h = _hmlp_call(x_pad, w1, b1r, w2, b2r)
    g1, dinv = _scale_call(degp, h)
    s1p = _seg_call(g1, eb, zeros_in)
    f1, g2 = _elem_call(h, dinv, s1p)
    s2p = _seg_call(g2, eb, zeros_in)
    return _final_call(h, f1, dinv, s2p, w3, b3r, w4, b4r)


# speedup vs baseline: 1.0146x; 1.0146x over previous
import jax
import jax.numpy as jnp
from jax import lax
from jax.experimental import pallas as pl
from jax.experimental.pallas import tpu as pltpu
from jax.experimental.pallas import tpu_sc as plsc

N = 10000
E = 320000
H = 128
C_OUT = 2
N_PAD = 10240
E_PAD = 327680
EROWS = E_PAD // 128
NC, NS = 2, 16
ROWS_PER_TILE = N_PAD // NS
R2 = 320
R6 = 400


def _sc_mesh():
    return plsc.VectorSubcoreMesh(
        core_axis_name="c", subcore_axis_name="s",
        num_cores=NC, num_subcores=NS)


RPT = EROWS // NC // NS


def _deg_body(dst2d, ones_in, zeros_in, degp, dstall, ones, y, sem):
    c = lax.axis_index("c")
    s = lax.axis_index("s")
    wid = c * NS + s
    pltpu.sync_copy(dst2d.at[pl.ds(wid * RPT, RPT)], dstall)
    pltpu.sync_copy(zeros_in, ones)
    base_r = s * ROWS_PER_TILE
    for k in range(ROWS_PER_TILE // 128):
        pltpu.sync_copy(ones, y.at[pl.ds(base_r + k * 128, 128)])
    pltpu.sync_copy(ones_in, ones)
    plsc.subcore_barrier()

    def step(j, carry):
        pltpu.sync_copy(ones, y.at[dstall.at[j]], add=True)
        return carry

    lax.fori_loop(0, RPT, step, 0)
    plsc.subcore_barrier()
    for k in range(ROWS_PER_TILE // 128):
        r = base_r + k * 128
        pltpu.sync_copy(y.at[pl.ds(r, 128)], degp.at[c, pl.ds(r, 128)])


_deg_call = pl.kernel(
    _deg_body,
    out_type=jax.ShapeDtypeStruct((NC, N_PAD, H), jnp.float32),
    mesh=_sc_mesh(),
    scratch_types=[
        pltpu.VMEM((RPT, 128), jnp.int32),
        pltpu.VMEM((128, H), jnp.float32),
        pltpu.VMEM_SHARED((N_PAD, H), jnp.float32),
        pltpu.SemaphoreType.DMA,
    ],
)


NBUF = 2


def _seg_body(g, eb, sout, ibuf, rows, y, gsem, ssem):
    c = lax.axis_index("c")
    s = lax.axis_index("s")
    wid = c * NS + s

    def zrow(i, carry):
        for k in range(8):
            rows[0, i, pl.ds(k * 16, 16)] = jnp.zeros((16,), jnp.float32)
        return carry

    lax.fori_loop(0, 128, zrow, 0)
    base_r = s * ROWS_PER_TILE
    for k in range(ROWS_PER_TILE // 128):
        pltpu.sync_copy(rows.at[0], y.at[pl.ds(base_r + k * 128, 128)])
    plsc.subcore_barrier()

    nbody = RPT // 8

    def mblock(m, carry):
        pltpu.sync_copy(eb.at[wid * nbody + m], ibuf)
        gcp = [None] * 8
        scp = [None] * 8
        gcp[0] = pltpu.async_copy(g.at[ibuf.at[0]], rows.at[0], gsem.at[0])
        for j in range(8):
            b = j % NBUF
            gcp[j].wait()
            scp[j] = pltpu.async_copy(
                rows.at[b], y.at[ibuf.at[8 + j]], ssem.at[b], add=True)
            if j < 7:
                bn = (j + 1) % NBUF
                if j - (NBUF - 1) >= 0:
                    scp[j - (NBUF - 1)].wait()
                gcp[j + 1] = pltpu.async_copy(
                    g.at[ibuf.at[j + 1]], rows.at[bn], gsem.at[bn])
        for j in range(8 - NBUF, 8):
            scp[j].wait()
        return carry

    lax.fori_loop(0, nbody, mblock, 0)
    plsc.subcore_barrier()
    for k in range(ROWS_PER_TILE // 128):
        r = base_r + k * 128
        pltpu.sync_copy(y.at[pl.ds(r, 128)], sout.at[c, pl.ds(r, 128)])


_seg_call = pl.kernel(
    _seg_body,
    out_type=jax.ShapeDtypeStruct((NC, N_PAD, H), jnp.float32),
    mesh=_sc_mesh(),
    scratch_types=[
        pltpu.VMEM((16, 128), jnp.int32),
        pltpu.VMEM((NBUF, 128, H), jnp.float32),
        pltpu.VMEM_SHARED((N_PAD, H), jnp.float32),
        pltpu.SemaphoreType.DMA((NBUF,)),
        pltpu.SemaphoreType.DMA((NBUF,)),
    ],
)


def _hmlp_body(x_ref, w1_ref, b1_ref, w2_ref, b2_ref, h_ref):
    x = x_ref[...]
    h1 = jnp.maximum(
        jnp.dot(x, w1_ref[...], preferred_element_type=jnp.float32)
        + b1_ref[...], 0.0)
    h_ref[...] = jnp.maximum(
        jnp.dot(h1, w2_ref[...], preferred_element_type=jnp.float32)
        + b2_ref[...], 0.0)


def _hmlp_call(x_pad, w1, b1r, w2, b2r):
    nb = N_PAD // R2
    return pl.pallas_call(
        _hmlp_body,
        grid=(nb,),
        in_specs=[
            pl.BlockSpec((R2, H), lambda i: (i, 0)),
            pl.BlockSpec((H, H), lambda i: (0, 0)),
            pl.BlockSpec((1, H), lambda i: (0, 0)),
            pl.BlockSpec((H, H), lambda i: (0, 0)),
            pl.BlockSpec((1, H), lambda i: (0, 0)),
        ],
        out_specs=pl.BlockSpec((R2, H), lambda i: (i, 0)),
        out_shape=jax.ShapeDtypeStruct((N_PAD, H), jnp.float32),
    )(x_pad, w1, b1r, w2, b2r)


def _scale_body(degp_ref, h_ref, g_ref, dinv_ref):
    i = pl.program_id(0)
    deg = (degp_ref[0, :, 0:1].astype(jnp.float32)
           + degp_ref[1, :, 0:1].astype(jnp.float32))
    dinv = lax.rsqrt(jnp.maximum(deg, 1.0))
    rows = i * R2 + lax.broadcasted_iota(jnp.int32, (R2, 1), 0)
    g = jnp.where(rows < N, h_ref[...] * dinv, 0.0)
    g_ref[...] = g
    dinv_ref[...] = dinv


def _scale_call(degp, h):
    nb = N_PAD // R2
    return pl.pallas_call(
        _scale_body,
        grid=(nb,),
        in_specs=[
            pl.BlockSpec((NC, R2, H), lambda i: (0, i, 0)),
            pl.BlockSpec((R2, H), lambda i: (i, 0)),
        ],
        out_specs=[
            pl.BlockSpec((R2, H), lambda i: (i, 0)),
            pl.BlockSpec((R2, 1), lambda i: (i, 0)),
        ],
        out_shape=[
            jax.ShapeDtypeStruct((N_PAD, H), jnp.float32),
            jax.ShapeDtypeStruct((N_PAD, 1), jnp.float32),
        ],
    )(degp, h)


def _elem_body(h_ref, dinv_ref, s1_ref, f1_ref, g2_ref):
    i = pl.program_id(0)
    h = h_ref[...]
    dinv = dinv_ref[...]
    s1 = s1_ref[0] + s1_ref[1]
    f1 = h - dinv * s1
    rows = i * R2 + lax.broadcasted_iota(jnp.int32, (R2, 1), 0)
    g2 = jnp.where(rows < N, f1 * dinv, 0.0)
    f1_ref[...] = f1
    g2_ref[...] = g2


def _elem_call(h, dinv, s1p):
    nb = N_PAD // R2
    return pl.pallas_call(
        _elem_body,
        grid=(nb,),
        in_specs=[
            pl.BlockSpec((R2, H), lambda i: (i, 0)),
            pl.BlockSpec((R2, 1), lambda i: (i, 0)),
            pl.BlockSpec((NC, R2, H), lambda i: (0, i, 0)),
        ],
        out_specs=[
            pl.BlockSpec((R2, H), lambda i: (i, 0)),
            pl.BlockSpec((R2, H), lambda i: (i, 0)),
        ],
        out_shape=[
            jax.ShapeDtypeStruct((N_PAD, H), jnp.float32),
            jax.ShapeDtypeStruct((N_PAD, H), jnp.float32),
        ],
    )(h, dinv, s1p)


def _final_body(h_ref, f1_ref, dinv_ref, s2_ref, w3_ref, b3_ref, w4_ref,
                b4_ref, o_ref):
    h = h_ref[...]
    f1 = f1_ref[...]
    dinv = dinv_ref[...]
    s2 = s2_ref[0] + s2_ref[1]
    f2 = f1 - dinv * s2
    t0 = 3.0 * h - 3.0 * f1 + 0.75 * f2
    t1 = 3.0 * f1 - 1.5 * f2
    t2 = 0.75 * f2
    hf = jnp.concatenate([t0, t1, t2], axis=1)
    z = jnp.maximum(
        jnp.dot(hf, w3_ref[...], preferred_element_type=jnp.float32)
        + b3_ref[...], 0.0)
    o_ref[...] = (
        jnp.dot(z, w4_ref[...], preferred_element_type=jnp.float32)
        + b4_ref[...])


def _final_call(h, f1, dinv, s2p, w3, b3r, w4, b4r):
    nb = N // R6
    return pl.pallas_call(
        _final_body,
        grid=(nb,),
        in_specs=[
            pl.BlockSpec((R6, H), lambda i: (i, 0)),
            pl.BlockSpec((R6, H), lambda i: (i, 0)),
            pl.BlockSpec((R6, 1), lambda i: (i, 0)),
            pl.BlockSpec((NC, R6, H), lambda i: (0, i, 0)),
            pl.BlockSpec((3 * H, H), lambda i: (0, 0)),
            pl.BlockSpec((1, H), lambda i: (0, 0)),
            pl.BlockSpec((H, C_OUT), lambda i: (0, 0)),
            pl.BlockSpec((1, C_OUT), lambda i: (0, 0)),
        ],
        out_specs=pl.BlockSpec((R6, C_OUT), lambda i: (i, 0)),
        out_shape=jax.ShapeDtypeStruct((N, C_OUT), jnp.float32),
    )(h, f1, dinv, s2p, w3, b3r, w4, b4r)


def kernel(in_feat, edge_index, w1, b1, w2, b2, w3, b3, w4, b4):
    x_pad = jnp.pad(in_feat, ((0, N_PAD - N), (0, 0)))
    nw = NC * NS
    padw = (E_PAD - E) // nw
    pad = jnp.broadcast_to(
        jnp.arange(N, N + padw, dtype=jnp.int32), (nw, padw))
    src_w = jnp.concatenate([edge_index[0].reshape(nw, E // nw), pad], 1)
    dst_w = jnp.concatenate([edge_index[1].reshape(nw, E // nw), pad], 1)
    dst2d = dst_w.reshape(EROWS, 128)
    src4 = src_w.reshape(nw, RPT // 8, 8, 128)
    dst4 = dst_w.reshape(nw, RPT // 8, 8, 128)
    eb = jnp.concatenate([src4, dst4], axis=2).reshape(
        nw * (RPT // 8), 16, 128)
    b1r = b1.reshape(1, H)
    b2r = b2.reshape(1, H)
    b3r = b3.reshape(1, H)
    b4r = b4.reshape(1, C_OUT)

    ones_in = jnp.ones((128, H), dtype=jnp.float32)
    zeros_in = jnp.zeros((128, H), dtype=jnp.float32)
    degp = _deg_call(dst2d, ones_in, zeros_in)
    h = _hmlp_call(x_pad, w1, b1r, w2, b2r)
    g1, dinv = _scale_call(degp, h)
    s1p = _seg_call(g1, eb)
    f1, g2 = _elem_call(h, dinv, s1p)
    s2p = _seg_call(g2, eb)
    return _final_call(h, f1, dinv, s2p, w3, b3r, w4, b4r)


# speedup vs baseline: 1.0183x; 1.0037x over previous
import jax
import jax.numpy as jnp
from jax import lax
from jax.experimental import pallas as pl
from jax.experimental.pallas import tpu as pltpu
from jax.experimental.pallas import tpu_sc as plsc

N = 10000
E = 320000
H = 128
C_OUT = 2
N_PAD = 10240
E_PAD = 327680
EROWS = E_PAD // 128
NC, NS = 2, 16
ROWS_PER_TILE = N_PAD // NS
R2 = 320
R6 = 400


def _sc_mesh():
    return plsc.VectorSubcoreMesh(
        core_axis_name="c", subcore_axis_name="s",
        num_cores=NC, num_subcores=NS)


RPT = EROWS // NC // NS


def _deg_body(dst2d, ones_in, zeros_in, degp, dstall, ones, y, sem):
    c = lax.axis_index("c")
    s = lax.axis_index("s")
    wid = c * NS + s
    pltpu.sync_copy(dst2d.at[pl.ds(wid * RPT, RPT)], dstall)
    pltpu.sync_copy(zeros_in, ones)
    base_r = s * ROWS_PER_TILE
    for k in range(ROWS_PER_TILE // 128):
        pltpu.sync_copy(ones, y.at[pl.ds(base_r + k * 128, 128)])
    pltpu.sync_copy(ones_in, ones)
    plsc.subcore_barrier()

    def step(j, carry):
        pltpu.sync_copy(ones, y.at[dstall.at[j]], add=True)
        return carry

    lax.fori_loop(0, RPT, step, 0)
    plsc.subcore_barrier()
    for k in range(ROWS_PER_TILE // 128):
        r = base_r + k * 128
        pltpu.sync_copy(y.at[pl.ds(r, 128)], degp.at[c, pl.ds(r, 128)])


_deg_call = pl.kernel(
    _deg_body,
    out_type=jax.ShapeDtypeStruct((NC, N_PAD, H), jnp.float32),
    mesh=_sc_mesh(),
    scratch_types=[
        pltpu.VMEM((RPT, 128), jnp.int32),
        pltpu.VMEM((128, H), jnp.float32),
        pltpu.VMEM_SHARED((N_PAD, H), jnp.float32),
        pltpu.SemaphoreType.DMA,
    ],
)


NBUF = 2


def _seg_body(g, eb, sout, ibuf, rows, y, gsem, ssem):
    c = lax.axis_index("c")
    s = lax.axis_index("s")
    wid = c * NS + s

    def zrow(i, carry):
        for k in range(8):
            rows[0, i, pl.ds(k * 16, 16)] = jnp.zeros((16,), jnp.float32)
        return carry

    lax.fori_loop(0, 128, zrow, 0)
    base_r = s * ROWS_PER_TILE
    for k in range(ROWS_PER_TILE // 128):
        pltpu.sync_copy(rows.at[0], y.at[pl.ds(base_r + k * 128, 128)])
    plsc.subcore_barrier()

    nbody = RPT // 8

    def mblock(m, carry):
        pltpu.sync_copy(eb.at[wid * nbody + m], ibuf)
        gcp = [None] * 8
        scp = [None] * 8
        gcp[0] = pltpu.async_copy(g.at[ibuf.at[0]], rows.at[0], gsem.at[0])
        for j in range(8):
            b = j % NBUF
            gcp[j].wait()
            scp[j] = pltpu.async_copy(
                rows.at[b], y.at[ibuf.at[8 + j]], ssem.at[b], add=True)
            if j < 7:
                bn = (j + 1) % NBUF
                if j - (NBUF - 1) >= 0:
                    scp[j - (NBUF - 1)].wait()
                gcp[j + 1] = pltpu.async_copy(
                    g.at[ibuf.at[j + 1]], rows.at[bn], gsem.at[bn])
        for j in range(8 - NBUF, 8):
            scp[j].wait()
        return carry

    lax.fori_loop(0, nbody, mblock, 0)
    plsc.subcore_barrier()
    for k in range(ROWS_PER_TILE // 128):
        r = base_r + k * 128
        pltpu.sync_copy(y.at[pl.ds(r, 128)], sout.at[c, pl.ds(r, 128)])


_seg_call = pl.kernel(
    _seg_body,
    out_type=jax.ShapeDtypeStruct((NC, N_PAD, H), jnp.float32),
    mesh=_sc_mesh(),
    scratch_types=[
        pltpu.VMEM((16, 128), jnp.int32),
        pltpu.VMEM((NBUF, 128, H), jnp.float32),
        pltpu.VMEM_SHARED((N_PAD, H), jnp.float32),
        pltpu.SemaphoreType.DMA((NBUF,)),
        pltpu.SemaphoreType.DMA((NBUF,)),
    ],
)


def _hmlp_body(x_ref, w1_ref, b1_ref, w2_ref, b2_ref, h_ref):
    x = x_ref[...]
    h1 = jnp.maximum(
        jnp.dot(x, w1_ref[...], preferred_element_type=jnp.float32)
        + b1_ref[...], 0.0)
    h_ref[...] = jnp.maximum(
        jnp.dot(h1, w2_ref[...], preferred_element_type=jnp.float32)
        + b2_ref[...], 0.0)


def _hmlp_call(x_pad, w1, b1r, w2, b2r):
    nb = N_PAD // R2
    return pl.pallas_call(
        _hmlp_body,
        grid=(nb,),
        in_specs=[
            pl.BlockSpec((R2, H), lambda i: (i, 0)),
            pl.BlockSpec((H, H), lambda i: (0, 0)),
            pl.BlockSpec((1, H), lambda i: (0, 0)),
            pl.BlockSpec((H, H), lambda i: (0, 0)),
            pl.BlockSpec((1, H), lambda i: (0, 0)),
        ],
        out_specs=pl.BlockSpec((R2, H), lambda i: (i, 0)),
        out_shape=jax.ShapeDtypeStruct((N_PAD, H), jnp.float32),
    )(x_pad, w1, b1r, w2, b2r)


def _scale_body(degp_ref, h_ref, g_ref, dinv_ref):
    i = pl.program_id(0)
    deg = (degp_ref[0, :, 0:1].astype(jnp.float32)
           + degp_ref[1, :, 0:1].astype(jnp.float32))
    dinv = lax.rsqrt(jnp.maximum(deg, 1.0))
    rows = i * R2 + lax.broadcasted_iota(jnp.int32, (R2, 1), 0)
    g = jnp.where(rows < N, h_ref[...] * dinv, 0.0)
    g_ref[...] = g
    dinv_ref[...] = dinv


def _scale_call(degp, h):
    nb = N_PAD // R2
    return pl.pallas_call(
        _scale_body,
        grid=(nb,),
        in_specs=[
            pl.BlockSpec((NC, R2, H), lambda i: (0, i, 0)),
            pl.BlockSpec((R2, H), lambda i: (i, 0)),
        ],
        out_specs=[
            pl.BlockSpec((R2, H), lambda i: (i, 0)),
            pl.BlockSpec((R2, 1), lambda i: (i, 0)),
        ],
        out_shape=[
            jax.ShapeDtypeStruct((N_PAD, H), jnp.float32),
            jax.ShapeDtypeStruct((N_PAD, 1), jnp.float32),
        ],
    )(degp, h)


def _elem_body(h_ref, dinv_ref, s1_ref, g2_ref):
    i = pl.program_id(0)
    h = h_ref[...]
    dinv = dinv_ref[...]
    s1 = s1_ref[0] + s1_ref[1]
    f1 = h - dinv * s1
    rows = i * R2 + lax.broadcasted_iota(jnp.int32, (R2, 1), 0)
    g2_ref[...] = jnp.where(rows < N, f1 * dinv, 0.0)


def _elem_call(h, dinv, s1p):
    nb = N_PAD // R2
    return pl.pallas_call(
        _elem_body,
        grid=(nb,),
        in_specs=[
            pl.BlockSpec((R2, H), lambda i: (i, 0)),
            pl.BlockSpec((R2, 1), lambda i: (i, 0)),
            pl.BlockSpec((NC, R2, H), lambda i: (0, i, 0)),
        ],
        out_specs=pl.BlockSpec((R2, H), lambda i: (i, 0)),
        out_shape=jax.ShapeDtypeStruct((N_PAD, H), jnp.float32),
    )(h, dinv, s1p)


def _final_body(h_ref, dinv_ref, s1_ref, s2_ref, w3_ref, b3_ref, w4_ref,
                b4_ref, o_ref):
    h = h_ref[...]
    dinv = dinv_ref[...]
    f1 = h - dinv * (s1_ref[0] + s1_ref[1])
    s2 = s2_ref[0] + s2_ref[1]
    f2 = f1 - dinv * s2
    t0 = 3.0 * h - 3.0 * f1 + 0.75 * f2
    t1 = 3.0 * f1 - 1.5 * f2
    t2 = 0.75 * f2
    hf = jnp.concatenate([t0, t1, t2], axis=1)
    z = jnp.maximum(
        jnp.dot(hf, w3_ref[...], preferred_element_type=jnp.float32)
        + b3_ref[...], 0.0)
    o_ref[...] = (
        jnp.dot(z, w4_ref[...], preferred_element_type=jnp.float32)
        + b4_ref[...])


def _final_call(h, dinv, s1p, s2p, w3, b3r, w4, b4r):
    nb = N // R6
    return pl.pallas_call(
        _final_body,
        grid=(nb,),
        in_specs=[
            pl.BlockSpec((R6, H), lambda i: (i, 0)),
            pl.BlockSpec((R6, 1), lambda i: (i, 0)),
            pl.BlockSpec((NC, R6, H), lambda i: (0, i, 0)),
            pl.BlockSpec((NC, R6, H), lambda i: (0, i, 0)),
            pl.BlockSpec((3 * H, H), lambda i: (0, 0)),
            pl.BlockSpec((1, H), lambda i: (0, 0)),
            pl.BlockSpec((H, C_OUT), lambda i: (0, 0)),
            pl.BlockSpec((1, C_OUT), lambda i: (0, 0)),
        ],
        out_specs=pl.BlockSpec((R6, C_OUT), lambda i: (i, 0)),
        out_shape=jax.ShapeDtypeStruct((N, C_OUT), jnp.float32),
    )(h, dinv, s1p, s2p, w3, b3r, w4, b4r)


def kernel(in_feat, edge_index, w1, b1, w2, b2, w3, b3, w4, b4):
    x_pad = jnp.pad(in_feat, ((0, N_PAD - N), (0, 0)))
    nw = NC * NS
    padw = (E_PAD - E) // nw
    pad = jnp.broadcast_to(
        jnp.arange(N, N + padw, dtype=jnp.int32), (nw, padw))
    src_w = jnp.concatenate([edge_index[0].reshape(nw, E // nw), pad], 1)
    dst_w = jnp.concatenate([edge_index[1].reshape(nw, E // nw), pad], 1)
    dst2d = dst_w.reshape(EROWS, 128)
    src4 = src_w.reshape(nw, RPT // 8, 8, 128)
    dst4 = dst_w.reshape(nw, RPT // 8, 8, 128)
    eb = jnp.concatenate([src4, dst4], axis=2).reshape(
        nw * (RPT // 8), 16, 128)
    b1r = b1.reshape(1, H)
    b2r = b2.reshape(1, H)
    b3r = b3.reshape(1, H)
    b4r = b4.reshape(1, C_OUT)

    ones_in = jnp.ones((128, H), dtype=jnp.float32)
    zeros_in = jnp.zeros((128, H), dtype=jnp.float32)
    degp = _deg_call(dst2d, ones_in, zeros_in)
    h = _hmlp_call(x_pad, w1, b1r, w2, b2r)
    g1, dinv = _scale_call(degp, h)
    s1p = _seg_call(g1, eb)
    g2 = _elem_call(h, dinv, s1p)
    s2p = _seg_call(g2, eb)
    return _final_call(h, dinv, s1p, s2p, w3, b3r, w4, b4r)


# speedup vs baseline: 1.0188x; 1.0005x over previous
import jax
import jax.numpy as jnp
from jax import lax
from jax.experimental import pallas as pl
from jax.experimental.pallas import tpu as pltpu
from jax.experimental.pallas import tpu_sc as plsc

N = 10000
E = 320000
H = 128
C_OUT = 2
N_PAD = 10240
E_PAD = 327680
EROWS = E_PAD // 128
NC, NS = 2, 16
ROWS_PER_TILE = N_PAD // NS
R2 = 320
R6 = 400


def _sc_mesh():
    return plsc.VectorSubcoreMesh(
        core_axis_name="c", subcore_axis_name="s",
        num_cores=NC, num_subcores=NS)


RPT = EROWS // NC // NS


def _deg_body(dst2d, ones_in, zeros_in, degp, dstall, ones, y, sem):
    c = lax.axis_index("c")
    s = lax.axis_index("s")
    wid = c * NS + s
    pltpu.sync_copy(dst2d.at[pl.ds(wid * RPT, RPT)], dstall)
    pltpu.sync_copy(zeros_in, ones)
    base_r = s * ROWS_PER_TILE
    for k in range(ROWS_PER_TILE // 128):
        pltpu.sync_copy(ones, y.at[pl.ds(base_r + k * 128, 128)])
    pltpu.sync_copy(ones_in, ones)
    plsc.subcore_barrier()

    for j in range(4):
        pltpu.async_copy(ones, y.at[dstall.at[j]], sem, add=True)

    def step(m, carry):
        e = 4 + 2 * m
        pltpu.make_async_copy(degp.at[c, pl.ds(0, 128)], ones, sem).wait()
        pltpu.make_async_copy(degp.at[c, pl.ds(0, 128)], ones, sem).wait()
        pltpu.async_copy(ones, y.at[dstall.at[e]], sem, add=True)
        pltpu.async_copy(ones, y.at[dstall.at[e + 1]], sem, add=True)
        return carry

    lax.fori_loop(0, (RPT - 4) // 2, step, 0)
    for j in range(4):
        pltpu.make_async_copy(degp.at[c, pl.ds(0, 128)], ones, sem).wait()
    plsc.subcore_barrier()
    for k in range(ROWS_PER_TILE // 128):
        r = base_r + k * 128
        pltpu.sync_copy(y.at[pl.ds(r, 128)], degp.at[c, pl.ds(r, 128)])


_deg_call = pl.kernel(
    _deg_body,
    out_type=jax.ShapeDtypeStruct((NC, N_PAD, H), jnp.float32),
    mesh=_sc_mesh(),
    scratch_types=[
        pltpu.VMEM((RPT, 128), jnp.int32),
        pltpu.VMEM((128, H), jnp.float32),
        pltpu.VMEM_SHARED((N_PAD, H), jnp.float32),
        pltpu.SemaphoreType.DMA,
    ],
)


NBUF = 2


def _seg_body(g, eb, sout, ibuf, rows, y, gsem, ssem):
    c = lax.axis_index("c")
    s = lax.axis_index("s")
    wid = c * NS + s

    def zrow(i, carry):
        for k in range(8):
            rows[0, i, pl.ds(k * 16, 16)] = jnp.zeros((16,), jnp.float32)
        return carry

    lax.fori_loop(0, 128, zrow, 0)
    base_r = s * ROWS_PER_TILE
    for k in range(ROWS_PER_TILE // 128):
        pltpu.sync_copy(rows.at[0], y.at[pl.ds(base_r + k * 128, 128)])
    plsc.subcore_barrier()

    nbody = RPT // 8

    def mblock(m, carry):
        pltpu.sync_copy(eb.at[wid * nbody + m], ibuf)
        gcp = [None] * 8
        scp = [None] * 8
        gcp[0] = pltpu.async_copy(g.at[ibuf.at[0]], rows.at[0], gsem.at[0])
        for j in range(8):
            b = j % NBUF
            gcp[j].wait()
            scp[j] = pltpu.async_copy(
                rows.at[b], y.at[ibuf.at[8 + j]], ssem.at[b], add=True)
            if j < 7:
                bn = (j + 1) % NBUF
                if j - (NBUF - 1) >= 0:
                    scp[j - (NBUF - 1)].wait()
                gcp[j + 1] = pltpu.async_copy(
                    g.at[ibuf.at[j + 1]], rows.at[bn], gsem.at[bn])
        for j in range(8 - NBUF, 8):
            scp[j].wait()
        return carry

    lax.fori_loop(0, nbody, mblock, 0)
    plsc.subcore_barrier()
    for k in range(ROWS_PER_TILE // 128):
        r = base_r + k * 128
        pltpu.sync_copy(y.at[pl.ds(r, 128)], sout.at[c, pl.ds(r, 128)])


_seg_call = pl.kernel(
    _seg_body,
    out_type=jax.ShapeDtypeStruct((NC, N_PAD, H), jnp.float32),
    mesh=_sc_mesh(),
    scratch_types=[
        pltpu.VMEM((16, 128), jnp.int32),
        pltpu.VMEM((NBUF, 128, H), jnp.float32),
        pltpu.VMEM_SHARED((N_PAD, H), jnp.float32),
        pltpu.SemaphoreType.DMA((NBUF,)),
        pltpu.SemaphoreType.DMA((NBUF,)),
    ],
)


def _hmlp_body(x_ref, w1_ref, b1_ref, w2_ref, b2_ref, h_ref):
    x = x_ref[...]
    h1 = jnp.maximum(
        jnp.dot(x, w1_ref[...], preferred_element_type=jnp.float32)
        + b1_ref[...], 0.0)
    h_ref[...] = jnp.maximum(
        jnp.dot(h1, w2_ref[...], preferred_element_type=jnp.float32)
        + b2_ref[...], 0.0)


def _hmlp_call(x_pad, w1, b1r, w2, b2r):
    nb = N_PAD // R2
    return pl.pallas_call(
        _hmlp_body,
        grid=(nb,),
        in_specs=[
            pl.BlockSpec((R2, H), lambda i: (i, 0)),
            pl.BlockSpec((H, H), lambda i: (0, 0)),
            pl.BlockSpec((1, H), lambda i: (0, 0)),
            pl.BlockSpec((H, H), lambda i: (0, 0)),
            pl.BlockSpec((1, H), lambda i: (0, 0)),
        ],
        out_specs=pl.BlockSpec((R2, H), lambda i: (i, 0)),
        out_shape=jax.ShapeDtypeStruct((N_PAD, H), jnp.float32),
    )(x_pad, w1, b1r, w2, b2r)


def _scale_body(degp_ref, h_ref, g_ref, dinv_ref):
    i = pl.program_id(0)
    deg = (degp_ref[0, :, 0:1].astype(jnp.float32)
           + degp_ref[1, :, 0:1].astype(jnp.float32))
    dinv = lax.rsqrt(jnp.maximum(deg, 1.0))
    rows = i * R2 + lax.broadcasted_iota(jnp.int32, (R2, 1), 0)
    g = jnp.where(rows < N, h_ref[...] * dinv, 0.0)
    g_ref[...] = g
    dinv_ref[...] = dinv


def _scale_call(degp, h):
    nb = N_PAD // R2
    return pl.pallas_call(
        _scale_body,
        grid=(nb,),
        in_specs=[
            pl.BlockSpec((NC, R2, H), lambda i: (0, i, 0)),
            pl.BlockSpec((R2, H), lambda i: (i, 0)),
        ],
        out_specs=[
            pl.BlockSpec((R2, H), lambda i: (i, 0)),
            pl.BlockSpec((R2, 1), lambda i: (i, 0)),
        ],
        out_shape=[
            jax.ShapeDtypeStruct((N_PAD, H), jnp.float32),
            jax.ShapeDtypeStruct((N_PAD, 1), jnp.float32),
        ],
    )(degp, h)


def _elem_body(h_ref, dinv_ref, s1_ref, g2_ref):
    i = pl.program_id(0)
    h = h_ref[...]
    dinv = dinv_ref[...]
    s1 = s1_ref[0] + s1_ref[1]
    f1 = h - dinv * s1
    rows = i * R2 + lax.broadcasted_iota(jnp.int32, (R2, 1), 0)
    g2_ref[...] = jnp.where(rows < N, f1 * dinv, 0.0)


def _elem_call(h, dinv, s1p):
    nb = N_PAD // R2
    return pl.pallas_call(
        _elem_body,
        grid=(nb,),
        in_specs=[
            pl.BlockSpec((R2, H), lambda i: (i, 0)),
            pl.BlockSpec((R2, 1), lambda i: (i, 0)),
            pl.BlockSpec((NC, R2, H), lambda i: (0, i, 0)),
        ],
        out_specs=pl.BlockSpec((R2, H), lambda i: (i, 0)),
        out_shape=jax.ShapeDtypeStruct((N_PAD, H), jnp.float32),
    )(h, dinv, s1p)


def _final_body(h_ref, dinv_ref, s1_ref, s2_ref, w3_ref, b3_ref, w4_ref,
                b4_ref, o_ref):
    h = h_ref[...]
    dinv = dinv_ref[...]
    f1 = h - dinv * (s1_ref[0] + s1_ref[1])
    s2 = s2_ref[0] + s2_ref[1]
    f2 = f1 - dinv * s2
    t0 = 3.0 * h - 3.0 * f1 + 0.75 * f2
    t1 = 3.0 * f1 - 1.5 * f2
    t2 = 0.75 * f2
    hf = jnp.concatenate([t0, t1, t2], axis=1)
    z = jnp.maximum(
        jnp.dot(hf, w3_ref[...], preferred_element_type=jnp.float32)
        + b3_ref[...], 0.0)
    o_ref[...] = (
        jnp.dot(z, w4_ref[...], preferred_element_type=jnp.float32)
        + b4_ref[...])


def _final_call(h, dinv, s1p, s2p, w3, b3r, w4, b4r):
    nb = N // R6
    return pl.pallas_call(
        _final_body,
        grid=(nb,),
        in_specs=[
            pl.BlockSpec((R6, H), lambda i: (i, 0)),
            pl.BlockSpec((R6, 1), lambda i: (i, 0)),
            pl.BlockSpec((NC, R6, H), lambda i: (0, i, 0)),
            pl.BlockSpec((NC, R6, H), lambda i: (0, i, 0)),
            pl.BlockSpec((3 * H, H), lambda i: (0, 0)),
            pl.BlockSpec((1, H), lambda i: (0, 0)),
            pl.BlockSpec((H, C_OUT), lambda i: (0, 0)),
            pl.BlockSpec((1, C_OUT), lambda i: (0, 0)),
        ],
        out_specs=pl.BlockSpec((R6, C_OUT), lambda i: (i, 0)),
        out_shape=jax.ShapeDtypeStruct((N, C_OUT), jnp.float32),
    )(h, dinv, s1p, s2p, w3, b3r, w4, b4r)


def kernel(in_feat, edge_index, w1, b1, w2, b2, w3, b3, w4, b4):
    x_pad = jnp.pad(in_feat, ((0, N_PAD - N), (0, 0)))
    nw = NC * NS
    padw = (E_PAD - E) // nw
    pad = jnp.broadcast_to(
        jnp.arange(N, N + padw, dtype=jnp.int32), (nw, padw))
    src_w = jnp.concatenate([edge_index[0].reshape(nw, E // nw), pad], 1)
    dst_w = jnp.concatenate([edge_index[1].reshape(nw, E // nw), pad], 1)
    dst2d = dst_w.reshape(EROWS, 128)
    src4 = src_w.reshape(nw, RPT // 8, 8, 128)
    dst4 = dst_w.reshape(nw, RPT // 8, 8, 128)
    eb = jnp.concatenate([src4, dst4], axis=2).reshape(
        nw * (RPT // 8), 16, 128)
    b1r = b1.reshape(1, H)
    b2r = b2.reshape(1, H)
    b3r = b3.reshape(1, H)
    b4r = b4.reshape(1, C_OUT)

    ones_in = jnp.ones((128, H), dtype=jnp.float32)
    zeros_in = jnp.zeros((128, H), dtype=jnp.float32)
    degp = _deg_call(dst2d, ones_in, zeros_in)
    h = _hmlp_call(x_pad, w1, b1r, w2, b2r)
    g1, dinv = _scale_call(degp, h)
    s1p = _seg_call(g1, eb)
    g2 = _elem_call(h, dinv, s1p)
    s2p = _seg_call(g2, eb)
    return _final_call(h, dinv, s1p, s2p, w3, b3r, w4, b4r)
